# Initial kernel scaffold; baseline (speedup 1.0000x reference)
import jax
import jax.numpy as jnp
from jax.experimental import pallas as pl


def kernel(token_positions, PE):
    raise NotImplementedError("write your pallas kernel here")


# speedup vs baseline: 1.9776x; 1.9776x over previous
import functools

import jax
import jax.numpy as jnp
from jax import lax
from jax.experimental import pallas as pl
from jax.experimental.pallas import tpu as pltpu
from jax.experimental.pallas import tpu_sc as plsc

D_MODEL = 1024
NC = 2
NS = 16
NW = NC * NS
K = 32
N_CHUNKS = 32


def _pe_gather(idx3, table):
    B = NW * N_CHUNKS * K
    mesh = plsc.VectorSubcoreMesh(core_axis_name="c", subcore_axis_name="s")

    @functools.partial(
        pl.kernel,
        mesh=mesh,
        out_type=jax.ShapeDtypeStruct((B, D_MODEL), jnp.float32),
        scratch_types=[
            pltpu.VMEM((N_CHUNKS, K), jnp.int32),
            pltpu.VMEM((K, D_MODEL), jnp.float32),
            pltpu.SemaphoreType.DMA,
        ],
    )
    def body(idx_hbm, table_hbm, out_hbm, idx_v, rows_v, sem):
        wid = lax.axis_index("s") * NC + lax.axis_index("c")
        base = wid * (N_CHUNKS * K)
        pltpu.sync_copy(idx_hbm.at[wid], idx_v)

        def chunk(c, carry):
            pltpu.async_copy(table_hbm.at[idx_v.at[c]], rows_v, sem).wait()
            pltpu.sync_copy(rows_v, out_hbm.at[pl.ds(base + c * K, K)])
            return carry

        lax.fori_loop(0, N_CHUNKS, chunk, 0)

    return body(idx3, table)


def kernel(token_positions, PE):
    idx3 = token_positions.reshape(NW, N_CHUNKS, K)
    out = _pe_gather(idx3, PE)
    return out.reshape(token_positions.shape + (D_MODEL,))


# speedup vs baseline: 2.3598x; 1.1933x over previous
import functools

import jax
import jax.numpy as jnp
from jax import lax
from jax.experimental import pallas as pl
from jax.experimental.pallas import tpu as pltpu
from jax.experimental.pallas import tpu_sc as plsc

D_MODEL = 1024
NC = 2
NS = 16
NW = NC * NS
K = 32
N_CHUNKS = 32


def _pe_gather(idx3, table):
    B = NW * N_CHUNKS * K
    mesh = plsc.VectorSubcoreMesh(core_axis_name="c", subcore_axis_name="s")

    @functools.partial(
        pl.kernel,
        mesh=mesh,
        out_type=jax.ShapeDtypeStruct((B, D_MODEL), jnp.float32),
        scratch_types=[
            pltpu.VMEM((N_CHUNKS, K), jnp.int32),
            pltpu.VMEM((K, D_MODEL), jnp.float32),
            pltpu.VMEM((K, D_MODEL), jnp.float32),
            pltpu.SemaphoreType.DMA,
            pltpu.SemaphoreType.DMA,
            pltpu.SemaphoreType.DMA,
            pltpu.SemaphoreType.DMA,
        ],
    )
    def body(idx_hbm, table_hbm, out_hbm, idx_v, buf0, buf1, g0, g1, w0, w1):
        wid = lax.axis_index("s") * NC + lax.axis_index("c")
        base = wid * (N_CHUNKS * K)
        pltpu.sync_copy(idx_hbm.at[wid], idx_v)

        bufs = (buf0, buf1)
        gsems = (g0, g1)
        wsems = (w0, w1)

        def gather(c, b):
            return pltpu.async_copy(table_hbm.at[idx_v.at[c]], bufs[b], gsems[b])

        def write(c, b):
            return pltpu.async_copy(
                bufs[b], out_hbm.at[pl.ds(base + c * K, K)], wsems[b])

        gather(0, 0)
        gather(1, 1)

        def pair(i, carry):
            for b in range(2):
                c = 2 * i + b
                pltpu.make_async_copy(
                    table_hbm.at[idx_v.at[c]], bufs[b], gsems[b]).wait()
                write(c, b)
                pltpu.make_async_copy(
                    bufs[b], out_hbm.at[pl.ds(base + c * K, K)], wsems[b]).wait()

                @pl.when(c + 2 < N_CHUNKS)
                def _():
                    gather(c + 2, b)

            return carry

        lax.fori_loop(0, N_CHUNKS // 2, pair, 0)

    return body(idx3, table)


def kernel(token_positions, PE):
    idx3 = token_positions.reshape(NW, N_CHUNKS, K)
    out = _pe_gather(idx3, PE)
    return out.reshape(token_positions.shape + (D_MODEL,))


# speedup vs baseline: 2.3716x; 1.0050x over previous
import functools

import jax
import jax.numpy as jnp
from jax import lax
from jax.experimental import pallas as pl
from jax.experimental.pallas import tpu as pltpu
from jax.experimental.pallas import tpu_sc as plsc

D_MODEL = 1024
NC = 2
NS = 16
NW = NC * NS
K = 16
N_CHUNKS = 64
NBUF = 4


def _pe_gather(idx3, table):
    B = NW * N_CHUNKS * K
    mesh = plsc.VectorSubcoreMesh(core_axis_name="c", subcore_axis_name="s")

    @functools.partial(
        pl.kernel,
        mesh=mesh,
        out_type=jax.ShapeDtypeStruct((B, D_MODEL), jnp.float32),
        scratch_types=(
            [pltpu.VMEM((N_CHUNKS, K), jnp.int32)]
            + [pltpu.VMEM((K, D_MODEL), jnp.float32) for _ in range(NBUF)]
            + [pltpu.SemaphoreType.DMA for _ in range(2 * NBUF)]
        ),
    )
    def body(idx_hbm, table_hbm, out_hbm, idx_v, *rest):
        bufs = rest[:NBUF]
        gsems = rest[NBUF:2 * NBUF]
        wsems = rest[2 * NBUF:]
        wid = lax.axis_index("s") * NC + lax.axis_index("c")
        base = wid * (N_CHUNKS * K)
        pltpu.sync_copy(idx_hbm.at[wid], idx_v)

        def gather(c, b):
            return pltpu.async_copy(table_hbm.at[idx_v.at[c]], bufs[b], gsems[b])

        def wait_gather(c, b):
            pltpu.make_async_copy(
                table_hbm.at[idx_v.at[c]], bufs[b], gsems[b]).wait()

        def write(c, b):
            return pltpu.async_copy(
                bufs[b], out_hbm.at[pl.ds(base + c * K, K)], wsems[b])

        def wait_write(c, b):
            pltpu.make_async_copy(
                bufs[b], out_hbm.at[pl.ds(base + c * K, K)], wsems[b]).wait()

        gather(0, 0)
        gather(1, 1)

        def ring(i, carry):
            for j in range(NBUF):
                c = NBUF * i + j
                b = j
                bn = (j + 2) % NBUF
                wait_gather(c, b)
                write(c, b)

                @pl.when(c >= 2)
                def _():
                    wait_write(c - 2, bn)

                @pl.when(c + 2 < N_CHUNKS)
                def _():
                    gather(c + 2, bn)

            return carry

        lax.fori_loop(0, N_CHUNKS // NBUF, ring, 0)
        wait_write(N_CHUNKS - 2, (N_CHUNKS - 2) % NBUF)
        wait_write(N_CHUNKS - 1, (N_CHUNKS - 1) % NBUF)

    return body(idx3, table)


def kernel(token_positions, PE):
    idx3 = token_positions.reshape(NW, N_CHUNKS, K)
    out = _pe_gather(idx3, PE)
    return out.reshape(token_positions.shape + (D_MODEL,))
